# baseline (speedup 1.0000x reference)
import functools

import jax
import jax.numpy as jnp
from jax import lax
from jax.experimental import pallas as pl
from jax.experimental.pallas import tpu as pltpu
from jax.experimental.pallas import tpu_sc as plsc

_B = 16384
_EMB = 64
_EPS = 1e-5


@functools.lru_cache(maxsize=None)
def _make_sc_gather():
    info = plsc.get_sparse_core_info()
    nc, ns = info.num_cores, info.num_subcores
    nw = nc * ns
    bpw = _B // nw

    mesh = plsc.VectorSubcoreMesh(core_axis_name="c", subcore_axis_name="s")

    def body(uid_hbm, iid_hbm, uemb_hbm, iemb_hbm, ub_hbm, ib_hbm,
             ue_out, ie_out, bias_out,
             uidx_v, iidx_v, urow_v, irow_v,
             uhi_v, ihi_v, ubr_v, ibr_v, bsum_v,
             s0, s1, s2, s3):
        wid = lax.axis_index("s") * nc + lax.axis_index("c")
        base = wid * bpw
        pltpu.sync_copy(uid_hbm.at[pl.ds(base, bpw)], uidx_v)
        pltpu.sync_copy(iid_hbm.at[pl.ds(base, bpw)], iidx_v)
        for k in range(bpw // 16):
            sl = pl.ds(k * 16, 16)
            uhi_v[sl] = uidx_v[sl] >> 4
            ihi_v[sl] = iidx_v[sl] >> 4
        cu = pltpu.async_copy(uemb_hbm.at[uidx_v], urow_v, s0)
        ci = pltpu.async_copy(iemb_hbm.at[iidx_v], irow_v, s1)
        cb0 = pltpu.async_copy(ub_hbm.at[uhi_v], ubr_v, s2)
        cb1 = pltpu.async_copy(ib_hbm.at[ihi_v], ibr_v, s3)
        cu.wait()
        pltpu.sync_copy(urow_v, ue_out.at[pl.ds(base, bpw)])
        ci.wait()
        pltpu.sync_copy(irow_v, ie_out.at[pl.ds(base, bpw)])
        cb0.wait()
        cb1.wait()
        rid = lax.iota(jnp.int32, 16)
        for k in range(bpw // 16):
            sl = pl.ds(k * 16, 16)
            r = rid + (k * 16)
            bu = plsc.load_gather(ubr_v, [r, uidx_v[sl] & 15])
            bi = plsc.load_gather(ibr_v, [r, iidx_v[sl] & 15])
            bsum_v[sl] = bu + bi
        pltpu.sync_copy(bsum_v, bias_out.at[pl.ds(base, bpw)])

    f32 = jnp.float32
    return pl.kernel(
        body,
        out_type=(
            jax.ShapeDtypeStruct((_B, _EMB), f32),
            jax.ShapeDtypeStruct((_B, _EMB), f32),
            jax.ShapeDtypeStruct((_B,), f32),
        ),
        mesh=mesh,
        compiler_params=pltpu.CompilerParams(
            use_tc_tiling_on_sc=False, needs_layout_passes=False),
        scratch_types=[
            pltpu.VMEM((bpw,), jnp.int32),
            pltpu.VMEM((bpw,), jnp.int32),
            pltpu.VMEM((bpw, _EMB), f32),
            pltpu.VMEM((bpw, _EMB), f32),
            pltpu.VMEM((bpw,), jnp.int32),
            pltpu.VMEM((bpw,), jnp.int32),
            pltpu.VMEM((bpw, 16), f32),
            pltpu.VMEM((bpw, 16), f32),
            pltpu.VMEM((bpw,), f32),
            pltpu.SemaphoreType.DMA,
            pltpu.SemaphoreType.DMA,
            pltpu.SemaphoreType.DMA,
            pltpu.SemaphoreType.DMA,
        ],
    )


_BM = 2048


def _mlp_body(ue_ref, ie_ref, bias_ref,
              w1a_ref, w1b_ref, b1_ref, s1_ref, be1_ref,
              w2_ref, b2_ref, s2_ref, be2_ref,
              w3_ref, b3_ref, s3_ref, be3_ref,
              w4_ref, b4_ref, out_ref):
    f32 = jnp.float32
    h = jnp.dot(ue_ref[...], w1a_ref[...], preferred_element_type=f32)
    h += jnp.dot(ie_ref[...], w1b_ref[...], preferred_element_type=f32)
    h = jnp.maximum(h + b1_ref[...], 0.0) * s1_ref[...] + be1_ref[...]
    h = jnp.dot(h, w2_ref[...], preferred_element_type=f32)
    h = jnp.maximum(h + b2_ref[...], 0.0) * s2_ref[...] + be2_ref[...]
    h = jnp.dot(h, w3_ref[...], preferred_element_type=f32)
    h = jnp.maximum(h + b3_ref[...], 0.0) * s3_ref[...] + be3_ref[...]
    out = jnp.sum(h * w4_ref[...], axis=1, keepdims=True)
    out_ref[...] = out + b4_ref[...] + bias_ref[...]


@functools.lru_cache(maxsize=None)
def _make_mlp():
    f32 = jnp.float32
    bspec_batch = lambda w: pl.BlockSpec((_BM, w), lambda i: (i, 0))
    bspec_full = lambda r, c: pl.BlockSpec((r, c), lambda i: (0, 0))
    in_specs = [
        bspec_batch(_EMB),
        bspec_batch(_EMB),
        bspec_batch(1),
        bspec_full(_EMB, 256),
        bspec_full(_EMB, 256),
        bspec_full(1, 256),
        bspec_full(1, 256),
        bspec_full(1, 256),
        bspec_full(256, 128),
        bspec_full(1, 128),
        bspec_full(1, 128),
        bspec_full(1, 128),
        bspec_full(128, 64),
        bspec_full(1, 64),
        bspec_full(1, 64),
        bspec_full(1, 64),
        bspec_full(1, 64),
        bspec_full(1, 1),
    ]
    return pl.pallas_call(
        _mlp_body,
        grid=(_B // _BM,),
        in_specs=in_specs,
        out_specs=pl.BlockSpec((_BM, 1), lambda i: (i, 0)),
        out_shape=jax.ShapeDtypeStruct((_B, 1), f32),
    )


def kernel(user_ids, item_ids, user_emb, item_emb, user_bias, item_bias,
           W1, b1, g1, be1, W2, b2, g2, be2, W3, b3, g3, be3, W4, b4):
    uid = user_ids.astype(jnp.int32)
    iid = item_ids.astype(jnp.int32)
    ub16 = user_bias.reshape(-1, 16)
    ib16 = item_bias.reshape(-1, 16)
    ue, ie, bias = _make_sc_gather()(
        uid, iid, user_emb, item_emb, ub16, ib16)

    inv = jnp.float32(1.0) / jnp.sqrt(jnp.float32(1.0 + _EPS))
    row = lambda v: v.reshape(1, -1)
    out = _make_mlp()(
        ue, ie, bias.reshape(-1, 1),
        W1[:_EMB], W1[_EMB:], row(b1), row(g1 * inv), row(be1),
        W2, row(b2), row(g2 * inv), row(be2),
        W3, row(b3), row(g3 * inv), row(be3),
        W4.reshape(1, -1), b4.reshape(1, 1),
    )
    return out[:, 0]
